# baseline (speedup 1.0000x reference)
import functools

import jax
import jax.numpy as jnp
from jax import lax
from jax.experimental import pallas as pl
from jax.experimental.pallas import tpu as pltpu
from jax.experimental.pallas import tpu_sc as plsc

EMB = 32
NTOK = 16384

NC = 2
NS = 16
NW = NC * NS
CHUNK = 128
CPW = NTOK // (NW * CHUNK)


def _emb_body(idx_hbm, table_hbm, out_hbm, idx_v, rows_v, sem):
    wid = lax.axis_index("s") * NC + lax.axis_index("c")
    base = wid * CPW
    pltpu.sync_copy(idx_hbm.at[pl.ds(base, CPW)], idx_v)
    copies = [
        pltpu.async_copy(table_hbm.at[idx_v.at[j]], rows_v.at[j], sem)
        for j in range(CPW)
    ]
    for c in copies:
        c.wait()
    pltpu.sync_copy(rows_v, out_hbm.at[pl.ds(base, CPW)])


_emb = functools.partial(
    pl.kernel,
    out_type=jax.ShapeDtypeStruct((NTOK // CHUNK, CHUNK, EMB), jnp.float32),
    mesh=plsc.VectorSubcoreMesh(core_axis_name="c", subcore_axis_name="s"),
    scratch_types=[
        pltpu.VMEM((CPW, CHUNK), jnp.int32),
        pltpu.VMEM((CPW, CHUNK, EMB), jnp.float32),
        pltpu.SemaphoreType.DMA,
    ],
    compiler_params=pltpu.CompilerParams(use_tc_tiling_on_sc=False),
)(_emb_body)


def kernel(tokens, weight, bias):
    idx = tokens.reshape(NTOK // CHUNK, CHUNK)
    out = _emb(idx, weight).reshape(NTOK, EMB)
    return (out, out)


# speedup vs baseline: 4.0340x; 4.0340x over previous
import functools

import jax
import jax.numpy as jnp
from jax import lax
from jax.experimental import pallas as pl
from jax.experimental.pallas import tpu as pltpu
from jax.experimental.pallas import tpu_sc as plsc

EMB = 32
NTOK = 16384
LANES = 16

NC = 2
NS = 16
NW = NC * NS
TPW = NTOK // NW
NBUF = 16


def _emb_body(tok_hbm, wt_hbm, out_hbm, tok_v, buf_v, out_v, sems):
    wid = lax.axis_index("s") * NC + lax.axis_index("c")
    base = wid * TPW
    pltpu.sync_copy(tok_hbm.at[pl.ds(base, TPW)], tok_v)

    e_lo = lax.iota(jnp.int32, LANES)
    e_hi = e_lo + LANES

    tv0 = tok_v[pl.ds(0, NBUF)]
    for b in range(NBUF):
        c0 = (tv0[b] // 128) * 128
        pltpu.async_copy(
            wt_hbm.at[:, pl.ds(c0, 128)], buf_v.at[b], sems.at[b]
        )

    def step(g, _):
        tv = tok_v[pl.ds(g * NBUF, NBUF)]
        tnext = tok_v[pl.ds(jnp.minimum(g + 1, TPW // NBUF - 1) * NBUF, NBUF)]
        for b in range(NBUF):
            i = g * NBUF + b
            pltpu.make_async_copy(
                wt_hbm.at[:, pl.ds(0, 128)], buf_v.at[b], sems.at[b]
            ).wait()
            t = tv[b]
            col = t - (t // 128) * 128
            col_v = jnp.full((LANES,), col, jnp.int32)
            pos_v = jnp.full((LANES,), i, jnp.int32)
            lo = plsc.load_gather(buf_v.at[b], [e_lo, col_v])
            hi = plsc.load_gather(buf_v.at[b], [e_hi, col_v])
            plsc.store_scatter(out_v, [e_lo, pos_v], lo)
            plsc.store_scatter(out_v, [e_hi, pos_v], hi)

            @pl.when(g + 1 < TPW // NBUF)
            def _():
                c0 = (tnext[b] // 128) * 128
                pltpu.async_copy(
                    wt_hbm.at[:, pl.ds(c0, 128)], buf_v.at[b], sems.at[b]
                )

        return _

    lax.fori_loop(0, TPW // NBUF, step, None, unroll=False)

    pltpu.sync_copy(out_v, out_hbm.at[:, pl.ds(base, TPW)])


_emb = functools.partial(
    pl.kernel,
    out_type=jax.ShapeDtypeStruct((EMB, NTOK), jnp.float32),
    mesh=plsc.VectorSubcoreMesh(core_axis_name="c", subcore_axis_name="s"),
    scratch_types=[
        pltpu.VMEM((TPW,), jnp.int32),
        pltpu.VMEM((NBUF, EMB, 128), jnp.float32),
        pltpu.VMEM((EMB, TPW), jnp.float32),
        pltpu.SemaphoreType.DMA((NBUF,)),
    ],
    compiler_params=pltpu.CompilerParams(needs_layout_passes=False),
)(_emb_body)


def kernel(tokens, weight, bias):
    out_t = _emb(tokens, weight.T)
    out = out_t.T
    return (out, out)


# speedup vs baseline: 4.0463x; 1.0031x over previous
import functools

import jax
import jax.numpy as jnp
from jax import lax
from jax.experimental import pallas as pl
from jax.experimental.pallas import tpu as pltpu
from jax.experimental.pallas import tpu_sc as plsc

EMB = 32
NTOK = 16384
LANES = 16

NC = 2
NS = 16
NW = NC * NS
TPW = NTOK // NW
NBUF = 16


def _emb_body(tok_hbm, wt_hbm, out_hbm, tok_v, buf_v, out_v, sems):
    wid = lax.axis_index("s") * NC + lax.axis_index("c")
    base = wid * TPW
    pltpu.sync_copy(tok_hbm.at[pl.ds(base, TPW)], tok_v)

    e_lo = lax.iota(jnp.int32, LANES)
    e_hi = e_lo + LANES

    def enqueue(t, b):
        c0 = (t // 128) * 128
        for rg in range(4):
            pltpu.async_copy(
                wt_hbm.at[pl.ds(rg * 8, 8), pl.ds(c0, 128)],
                buf_v.at[b, pl.ds(rg * 8, 8)],
                sems.at[b],
            )

    tv0 = tok_v[pl.ds(0, NBUF)]
    for b in range(NBUF):
        enqueue(tv0[b], b)

    def step(g, _):
        tv = tok_v[pl.ds(g * NBUF, NBUF)]
        tnext = tok_v[pl.ds(jnp.minimum(g + 1, TPW // NBUF - 1) * NBUF, NBUF)]
        for b in range(NBUF):
            i = g * NBUF + b
            pltpu.make_async_copy(
                wt_hbm.at[:, pl.ds(0, 128)], buf_v.at[b], sems.at[b]
            ).wait()
            t = tv[b]
            col = t - (t // 128) * 128
            col_v = jnp.full((LANES,), col, jnp.int32)
            pos_v = jnp.full((LANES,), i, jnp.int32)
            lo = plsc.load_gather(buf_v.at[b], [e_lo, col_v])
            hi = plsc.load_gather(buf_v.at[b], [e_hi, col_v])
            plsc.store_scatter(out_v, [e_lo, pos_v], lo)
            plsc.store_scatter(out_v, [e_hi, pos_v], hi)

            @pl.when(g + 1 < TPW // NBUF)
            def _():
                enqueue(tnext[b], b)

        return _

    lax.fori_loop(0, TPW // NBUF, step, None, unroll=False)

    pltpu.sync_copy(out_v, out_hbm.at[:, pl.ds(base, TPW)])


_emb = functools.partial(
    pl.kernel,
    out_type=jax.ShapeDtypeStruct((EMB, NTOK), jnp.float32),
    mesh=plsc.VectorSubcoreMesh(core_axis_name="c", subcore_axis_name="s"),
    scratch_types=[
        pltpu.VMEM((TPW,), jnp.int32),
        pltpu.VMEM((NBUF, EMB, 128), jnp.float32),
        pltpu.VMEM((EMB, TPW), jnp.float32),
        pltpu.SemaphoreType.DMA((NBUF,)),
    ],
    compiler_params=pltpu.CompilerParams(needs_layout_passes=False),
)(_emb_body)


def kernel(tokens, weight, bias):
    out_t = _emb(tokens, weight.T)
    out = out_t.T
    return (out, out)
